# baseline (speedup 1.0000x reference)
import jax
import jax.numpy as jnp
from jax.experimental import pallas as pl

NUM_CLUSTERS = 64
D_MODEL = 768
N_POINTS = 2048
TOP_K = 10

P_BLOCK = 128
GRID = N_POINTS // P_BLOCK
LANE = 128
N_CHUNKS = D_MODEL // LANE


def _body(xb_ref, c_ref, x64_ref, out_ref):
    xb = xb_ref[:, :]
    cols = []
    for j in range(NUM_CLUSTERS):
        diff = xb - c_ref[j, :][None, :]
        sq = diff * diff
        s = jnp.sum(sq[:, 0:LANE], axis=1, keepdims=True)
        for k in range(1, N_CHUNKS):
            s = s + jnp.sum(sq[:, k * LANE:(k + 1) * LANE], axis=1,
                            keepdims=True)
        cols.append(s)
    d2 = jnp.concatenate(cols, axis=1)
    d = jnp.sqrt(d2)

    lanes = jax.lax.broadcasted_iota(jnp.int32, (P_BLOCK, NUM_CLUSTERS), 1)
    x64 = x64_ref[:, :]
    for k in range(TOP_K):
        m = jnp.min(d, axis=1, keepdims=True)
        ismin = d == m
        idxk = jnp.min(jnp.where(ismin, lanes, NUM_CLUSTERS), axis=1,
                       keepdims=True)
        onehot = (lanes == idxk).astype(jnp.float32)
        out_ref[:, k, :] = jnp.dot(onehot, x64,
                                   preferred_element_type=jnp.float32,
                                   precision=jax.lax.Precision.HIGHEST)
        d = jnp.where(lanes == idxk, jnp.inf, d)


def kernel(x, cluster_centers):
    x64 = x[:NUM_CLUSTERS]
    out = pl.pallas_call(
        _body,
        grid=(GRID,),
        in_specs=[
            pl.BlockSpec((P_BLOCK, D_MODEL), lambda i: (i, 0)),
            pl.BlockSpec((NUM_CLUSTERS, D_MODEL), lambda i: (0, 0)),
            pl.BlockSpec((NUM_CLUSTERS, D_MODEL), lambda i: (0, 0)),
        ],
        out_specs=pl.BlockSpec((P_BLOCK, TOP_K, D_MODEL), lambda i: (i, 0, 0)),
        out_shape=jax.ShapeDtypeStruct((N_POINTS, TOP_K, D_MODEL),
                                       jnp.float32),
    )(x, cluster_centers, x64)
    return jnp.reshape(out, (1, N_POINTS * TOP_K, D_MODEL))


# speedup vs baseline: 1.0366x; 1.0366x over previous
import functools

import jax
import jax.numpy as jnp
from jax import lax
from jax.experimental import pallas as pl
from jax.experimental.pallas import tpu as pltpu
from jax.experimental.pallas import tpu_sc as plsc

NUM_CLUSTERS = 64
D_MODEL = 768
N_POINTS = 2048
TOP_K = 10

P_BLOCK = 128
GRID = N_POINTS // P_BLOCK
LANE = 128
N_CHUNKS = D_MODEL // LANE

OUT_ROWS = N_POINTS * TOP_K

NC = 2
NS = 16
NW = NC * NS
B_PER_W = OUT_ROWS // NW
SC_CHUNK = 64
N_SC_CHUNKS = B_PER_W // SC_CHUNK


def _topk_body(xb_ref, c_ref, idx_ref):
    xb = xb_ref[:, :]
    cols = []
    for j in range(NUM_CLUSTERS):
        diff = xb - c_ref[j, :][None, :]
        sq = diff * diff
        s = jnp.sum(sq[:, 0:LANE], axis=1, keepdims=True)
        for k in range(1, N_CHUNKS):
            s = s + jnp.sum(sq[:, k * LANE:(k + 1) * LANE], axis=1,
                            keepdims=True)
        cols.append(s)
    d2 = jnp.concatenate(cols, axis=1)
    d = jnp.sqrt(d2)

    lanes = jax.lax.broadcasted_iota(jnp.int32, (P_BLOCK, NUM_CLUSTERS), 1)
    for k in range(TOP_K):
        m = jnp.min(d, axis=1, keepdims=True)
        ismin = d == m
        idxk = jnp.min(jnp.where(ismin, lanes, NUM_CLUSTERS), axis=1,
                       keepdims=True)
        idx_ref[:, k:k + 1] = idxk
        d = jnp.where(lanes == idxk, jnp.inf, d)


_SC_MESH = plsc.VectorSubcoreMesh(core_axis_name="c", subcore_axis_name="s")


@functools.partial(
    pl.kernel,
    mesh=_SC_MESH,
    out_type=jax.ShapeDtypeStruct((OUT_ROWS, D_MODEL), jnp.float32),
    scratch_types=[
        pltpu.VMEM((B_PER_W,), jnp.int32),
        pltpu.VMEM((SC_CHUNK, D_MODEL), jnp.float32),
        pltpu.VMEM((SC_CHUNK, D_MODEL), jnp.float32),
        pltpu.SemaphoreType.DMA,
        pltpu.SemaphoreType.DMA,
    ],
)
def _sc_gather(table_hbm, idx_hbm, out_hbm, idx_v, rows_a, rows_b, sem_a,
               sem_b):
    wid = lax.axis_index("s") * NC + lax.axis_index("c")
    base = wid * B_PER_W
    pltpu.sync_copy(idx_hbm.at[pl.ds(base, B_PER_W)], idx_v)
    bufs = (rows_a, rows_b)
    sems = (sem_a, sem_b)
    copies = [None, None]
    copies[0] = pltpu.async_copy(
        table_hbm.at[idx_v.at[pl.ds(0, SC_CHUNK)]], bufs[0], sems[0])
    for c in range(N_SC_CHUNKS):
        nxt = c + 1
        if nxt < N_SC_CHUNKS:
            copies[nxt % 2] = pltpu.async_copy(
                table_hbm.at[idx_v.at[pl.ds(nxt * SC_CHUNK, SC_CHUNK)]],
                bufs[nxt % 2], sems[nxt % 2])
        copies[c % 2].wait()
        pltpu.sync_copy(bufs[c % 2],
                        out_hbm.at[pl.ds(base + c * SC_CHUNK, SC_CHUNK)])


def kernel(x, cluster_centers):
    indices = pl.pallas_call(
        _topk_body,
        grid=(GRID,),
        in_specs=[
            pl.BlockSpec((P_BLOCK, D_MODEL), lambda i: (i, 0)),
            pl.BlockSpec((NUM_CLUSTERS, D_MODEL), lambda i: (0, 0)),
        ],
        out_specs=pl.BlockSpec((P_BLOCK, TOP_K), lambda i: (i, 0)),
        out_shape=jax.ShapeDtypeStruct((N_POINTS, TOP_K), jnp.int32),
    )(x, cluster_centers)

    idx_flat = jnp.reshape(indices, (OUT_ROWS,))
    out = _sc_gather(x, idx_flat)
    return jnp.reshape(out, (1, OUT_ROWS, D_MODEL))


# speedup vs baseline: 1.2191x; 1.1760x over previous
import functools

import jax
import jax.numpy as jnp
from jax import lax
from jax.experimental import pallas as pl
from jax.experimental.pallas import tpu as pltpu
from jax.experimental.pallas import tpu_sc as plsc

NUM_CLUSTERS = 64
D_MODEL = 768
N_POINTS = 2048
TOP_K = 10

P_BLOCK = 128
GRID = N_POINTS // P_BLOCK
LANE = 128
N_CHUNKS = D_MODEL // LANE

OUT_ROWS = N_POINTS * TOP_K

NC = 2
NS = 16
NW = NC * NS
B_PER_W = OUT_ROWS // NW
SC_CHUNK = 64
N_SC_CHUNKS = B_PER_W // SC_CHUNK


N_CAND = 16


def _topk_body(xb_ref, c_ref, idx_ref):
    xb = xb_ref[:, :]
    c = c_ref[:, :]

    g_t = lax.dot_general(c, xb, (((1,), (1,)), ((), ())),
                          precision=lax.Precision.HIGHEST)
    cn2 = jnp.sum(c * c, axis=1)
    s_t = cn2[:, None] - 2.0 * g_t

    subl64 = lax.broadcasted_iota(jnp.int32, (NUM_CLUSTERS, P_BLOCK), 0)
    cand_rows = []
    for k in range(N_CAND):
        m = jnp.min(s_t, axis=0, keepdims=True)
        idxk = jnp.min(jnp.where(s_t == m, subl64, NUM_CLUSTERS), axis=0,
                       keepdims=True)
        cand_rows.append(idxk)
        s_t = jnp.where(subl64 == idxk, jnp.inf, s_t)

    dist_rows = []
    for k in range(N_CAND):
        onehot_t = (subl64 == cand_rows[k]).astype(jnp.float32)
        csel = lax.dot_general(onehot_t, c, (((0,), (0,)), ((), ())),
                               precision=lax.Precision.HIGHEST)
        diff = xb - csel
        sq = diff * diff
        s = jnp.sum(sq[:, 0:LANE], axis=1)
        for j in range(1, N_CHUNKS):
            s = s + jnp.sum(sq[:, j * LANE:(j + 1) * LANE], axis=1)
        dist_rows.append(s)
    d = jnp.sqrt(jnp.stack(dist_rows, axis=0))
    ids = jnp.concatenate(cand_rows, axis=0)

    for k in range(TOP_K):
        m = jnp.min(d, axis=0, keepdims=True)
        idk = jnp.min(jnp.where(d == m, ids, NUM_CLUSTERS), axis=0,
                      keepdims=True)
        idx_ref[k:k + 1, :] = idk
        d = jnp.where(ids == idk, jnp.inf, d)


_SC_MESH = plsc.VectorSubcoreMesh(core_axis_name="c", subcore_axis_name="s")


@functools.partial(
    pl.kernel,
    mesh=_SC_MESH,
    out_type=jax.ShapeDtypeStruct((OUT_ROWS, D_MODEL), jnp.float32),
    scratch_types=[
        pltpu.VMEM((B_PER_W,), jnp.int32),
        pltpu.VMEM((SC_CHUNK, D_MODEL), jnp.float32),
        pltpu.VMEM((SC_CHUNK, D_MODEL), jnp.float32),
        pltpu.SemaphoreType.DMA,
        pltpu.SemaphoreType.DMA,
    ],
)
def _sc_gather(table_hbm, idx_hbm, out_hbm, idx_v, rows_a, rows_b, sem_a,
               sem_b):
    wid = lax.axis_index("s") * NC + lax.axis_index("c")
    base = wid * B_PER_W
    pltpu.sync_copy(idx_hbm.at[pl.ds(base, B_PER_W)], idx_v)
    bufs = (rows_a, rows_b)
    sems = (sem_a, sem_b)
    copies = [None, None]
    copies[0] = pltpu.async_copy(
        table_hbm.at[idx_v.at[pl.ds(0, SC_CHUNK)]], bufs[0], sems[0])
    for c in range(N_SC_CHUNKS):
        nxt = c + 1
        if nxt < N_SC_CHUNKS:
            copies[nxt % 2] = pltpu.async_copy(
                table_hbm.at[idx_v.at[pl.ds(nxt * SC_CHUNK, SC_CHUNK)]],
                bufs[nxt % 2], sems[nxt % 2])
        copies[c % 2].wait()
        pltpu.sync_copy(bufs[c % 2],
                        out_hbm.at[pl.ds(base + c * SC_CHUNK, SC_CHUNK)])


def kernel(x, cluster_centers):
    indices = pl.pallas_call(
        _topk_body,
        grid=(GRID,),
        in_specs=[
            pl.BlockSpec((P_BLOCK, D_MODEL), lambda i: (i, 0)),
            pl.BlockSpec((NUM_CLUSTERS, D_MODEL), lambda i: (0, 0)),
        ],
        out_specs=pl.BlockSpec((TOP_K, P_BLOCK), lambda i: (0, i)),
        out_shape=jax.ShapeDtypeStruct((TOP_K, N_POINTS), jnp.int32),
    )(x, cluster_centers)

    idx_flat = jnp.reshape(jnp.transpose(indices), (OUT_ROWS,))
    out = _sc_gather(x, idx_flat)
    return jnp.reshape(out, (1, OUT_ROWS, D_MODEL))


# speedup vs baseline: 1.5234x; 1.2496x over previous
import functools

import jax
import jax.numpy as jnp
from jax import lax
from jax.experimental import pallas as pl
from jax.experimental.pallas import tpu as pltpu
from jax.experimental.pallas import tpu_sc as plsc

NUM_CLUSTERS = 64
D_MODEL = 768
N_POINTS = 2048
TOP_K = 10

P_BLOCK = 128
GRID = N_POINTS // P_BLOCK
LANE = 128
N_CHUNKS = D_MODEL // LANE

OUT_ROWS = N_POINTS * TOP_K

NC = 2
NS = 16
NW = NC * NS
B_PER_W = OUT_ROWS // NW
SC_CHUNK = 64
N_SC_CHUNKS = B_PER_W // SC_CHUNK


N_CAND = 16


def _topk_body(xb_ref, c_ref, idx_ref):
    xb = xb_ref[:, :]
    c = c_ref[:, :]

    g_t = lax.dot_general(c, xb, (((1,), (1,)), ((), ())),
                          precision=lax.Precision.HIGHEST)
    cn2 = jnp.sum(c * c, axis=1)
    s_t = cn2[:, None] - 2.0 * g_t

    subl64 = lax.broadcasted_iota(jnp.int32, (NUM_CLUSTERS, P_BLOCK), 0)
    cand_rows = []
    for k in range(N_CAND):
        m = jnp.min(s_t, axis=0, keepdims=True)
        idxk = jnp.min(jnp.where(s_t == m, subl64, NUM_CLUSTERS), axis=0,
                       keepdims=True)
        cand_rows.append(idxk)
        s_t = jnp.where(subl64 == idxk, jnp.inf, s_t)

    c_hi = c.astype(jnp.bfloat16)
    r1 = c - c_hi.astype(jnp.float32)
    c_mid = r1.astype(jnp.bfloat16)
    c_lo = (r1 - c_mid.astype(jnp.float32)).astype(jnp.bfloat16)
    dist_rows = []
    for k in range(N_CAND):
        onehot_t = (subl64 == cand_rows[k]).astype(jnp.bfloat16)
        sel = functools.partial(
            lax.dot_general,
            dimension_numbers=(((0,), (0,)), ((), ())),
            preferred_element_type=jnp.float32)
        csel = ((sel(onehot_t, c_hi) + sel(onehot_t, c_mid))
                + sel(onehot_t, c_lo))
        diff = xb - csel
        sq = diff * diff
        s = jnp.sum(sq[:, 0:LANE], axis=1)
        for j in range(1, N_CHUNKS):
            s = s + jnp.sum(sq[:, j * LANE:(j + 1) * LANE], axis=1)
        dist_rows.append(s)
    d = jnp.sqrt(jnp.stack(dist_rows, axis=0))
    ids = jnp.concatenate(cand_rows, axis=0)

    for k in range(TOP_K):
        m = jnp.min(d, axis=0, keepdims=True)
        idk = jnp.min(jnp.where(d == m, ids, NUM_CLUSTERS), axis=0,
                      keepdims=True)
        idx_ref[k:k + 1, :] = idk
        d = jnp.where(ids == idk, jnp.inf, d)


_SC_MESH = plsc.VectorSubcoreMesh(core_axis_name="c", subcore_axis_name="s")


@functools.partial(
    pl.kernel,
    mesh=_SC_MESH,
    out_type=jax.ShapeDtypeStruct((OUT_ROWS, D_MODEL), jnp.float32),
    scratch_types=[
        pltpu.VMEM((B_PER_W,), jnp.int32),
        pltpu.VMEM((SC_CHUNK, D_MODEL), jnp.float32),
        pltpu.VMEM((SC_CHUNK, D_MODEL), jnp.float32),
        pltpu.SemaphoreType.DMA,
        pltpu.SemaphoreType.DMA,
    ],
)
def _sc_gather(table_hbm, idx_hbm, out_hbm, idx_v, rows_a, rows_b, sem_a,
               sem_b):
    wid = lax.axis_index("s") * NC + lax.axis_index("c")
    base = wid * B_PER_W
    pltpu.sync_copy(idx_hbm.at[pl.ds(base, B_PER_W)], idx_v)
    bufs = (rows_a, rows_b)
    sems = (sem_a, sem_b)
    copies = [None, None]
    copies[0] = pltpu.async_copy(
        table_hbm.at[idx_v.at[pl.ds(0, SC_CHUNK)]], bufs[0], sems[0])
    for c in range(N_SC_CHUNKS):
        nxt = c + 1
        if nxt < N_SC_CHUNKS:
            copies[nxt % 2] = pltpu.async_copy(
                table_hbm.at[idx_v.at[pl.ds(nxt * SC_CHUNK, SC_CHUNK)]],
                bufs[nxt % 2], sems[nxt % 2])
        copies[c % 2].wait()
        pltpu.sync_copy(bufs[c % 2],
                        out_hbm.at[pl.ds(base + c * SC_CHUNK, SC_CHUNK)])


def kernel(x, cluster_centers):
    indices = pl.pallas_call(
        _topk_body,
        grid=(GRID,),
        in_specs=[
            pl.BlockSpec((P_BLOCK, D_MODEL), lambda i: (i, 0)),
            pl.BlockSpec((NUM_CLUSTERS, D_MODEL), lambda i: (0, 0)),
        ],
        out_specs=pl.BlockSpec((TOP_K, P_BLOCK), lambda i: (0, i)),
        out_shape=jax.ShapeDtypeStruct((TOP_K, N_POINTS), jnp.int32),
    )(x, cluster_centers)

    idx_flat = jnp.reshape(jnp.transpose(indices), (OUT_ROWS,))
    out = _sc_gather(x, idx_flat)
    return jnp.reshape(out, (1, OUT_ROWS, D_MODEL))


# speedup vs baseline: 1.6137x; 1.0593x over previous
import functools

import jax
import jax.numpy as jnp
from jax import lax
from jax.experimental import pallas as pl
from jax.experimental.pallas import tpu as pltpu
from jax.experimental.pallas import tpu_sc as plsc

NUM_CLUSTERS = 64
D_MODEL = 768
N_POINTS = 2048
TOP_K = 10

P_BLOCK = 128
GRID = N_POINTS // P_BLOCK
LANE = 128
N_CHUNKS = D_MODEL // LANE

OUT_ROWS = N_POINTS * TOP_K

NC = 2
NS = 16
NW = NC * NS
B_PER_W = OUT_ROWS // NW
SC_CHUNK = 64
N_SC_CHUNKS = B_PER_W // SC_CHUNK


N_CAND = 16


def _topk_body(xb_ref, c_ref, idx_ref):
    xb = xb_ref[:, :]
    c = c_ref[:, :]

    g_t = lax.dot_general(c, xb, (((1,), (1,)), ((), ())),
                          precision=lax.Precision.HIGHEST)
    cn2 = jnp.sum(c * c, axis=1)
    s_t = cn2[:, None] - 2.0 * g_t

    subl64 = lax.broadcasted_iota(jnp.int32, (NUM_CLUSTERS, P_BLOCK), 0)
    cand_rows = []
    for k in range(N_CAND):
        m = jnp.min(s_t, axis=0, keepdims=True)
        idxk = jnp.min(jnp.where(s_t == m, subl64, NUM_CLUSTERS), axis=0,
                       keepdims=True)
        cand_rows.append(idxk)
        s_t = jnp.where(subl64 == idxk, jnp.inf, s_t)

    c_hi = c.astype(jnp.bfloat16)
    r1 = c - c_hi.astype(jnp.float32)
    c_mid = r1.astype(jnp.bfloat16)
    c_lo = (r1 - c_mid.astype(jnp.float32)).astype(jnp.bfloat16)
    c3 = jnp.concatenate([c_hi, c_mid, c_lo], axis=0)
    subl192 = lax.broadcasted_iota(jnp.int32, (3 * NUM_CLUSTERS, P_BLOCK),
                                   0) & (NUM_CLUSTERS - 1)
    dist_rows = []
    for k in range(N_CAND):
        onehot3 = (subl192 == cand_rows[k]).astype(jnp.bfloat16)
        csel = lax.dot_general(onehot3, c3, (((0,), (0,)), ((), ())),
                               preferred_element_type=jnp.float32)
        diff = xb - csel
        sq = diff * diff
        s = jnp.sum(sq[:, 0:LANE], axis=1)
        for j in range(1, N_CHUNKS):
            s = s + jnp.sum(sq[:, j * LANE:(j + 1) * LANE], axis=1)
        dist_rows.append(s)
    d = jnp.sqrt(jnp.stack(dist_rows, axis=0))
    ids = jnp.concatenate(cand_rows, axis=0)

    for k in range(TOP_K):
        m = jnp.min(d, axis=0, keepdims=True)
        idk = jnp.min(jnp.where(d == m, ids, NUM_CLUSTERS), axis=0,
                      keepdims=True)
        idx_ref[k:k + 1, :] = idk
        d = jnp.where(ids == idk, jnp.inf, d)


_SC_MESH = plsc.VectorSubcoreMesh(core_axis_name="c", subcore_axis_name="s")


@functools.partial(
    pl.kernel,
    mesh=_SC_MESH,
    out_type=jax.ShapeDtypeStruct((OUT_ROWS, D_MODEL), jnp.float32),
    scratch_types=[
        pltpu.VMEM((B_PER_W,), jnp.int32),
        pltpu.VMEM((SC_CHUNK, D_MODEL), jnp.float32),
        pltpu.VMEM((SC_CHUNK, D_MODEL), jnp.float32),
        pltpu.SemaphoreType.DMA,
        pltpu.SemaphoreType.DMA,
    ],
)
def _sc_gather(table_hbm, idx_hbm, out_hbm, idx_v, rows_a, rows_b, sem_a,
               sem_b):
    wid = lax.axis_index("s") * NC + lax.axis_index("c")
    base = wid * B_PER_W
    pltpu.sync_copy(idx_hbm.at[pl.ds(base, B_PER_W)], idx_v)
    bufs = (rows_a, rows_b)
    sems = (sem_a, sem_b)
    copies = [None, None]
    copies[0] = pltpu.async_copy(
        table_hbm.at[idx_v.at[pl.ds(0, SC_CHUNK)]], bufs[0], sems[0])
    for c in range(N_SC_CHUNKS):
        nxt = c + 1
        if nxt < N_SC_CHUNKS:
            copies[nxt % 2] = pltpu.async_copy(
                table_hbm.at[idx_v.at[pl.ds(nxt * SC_CHUNK, SC_CHUNK)]],
                bufs[nxt % 2], sems[nxt % 2])
        copies[c % 2].wait()
        pltpu.sync_copy(bufs[c % 2],
                        out_hbm.at[pl.ds(base + c * SC_CHUNK, SC_CHUNK)])


def kernel(x, cluster_centers):
    indices = pl.pallas_call(
        _topk_body,
        grid=(GRID,),
        in_specs=[
            pl.BlockSpec((P_BLOCK, D_MODEL), lambda i: (i, 0)),
            pl.BlockSpec((NUM_CLUSTERS, D_MODEL), lambda i: (0, 0)),
        ],
        out_specs=pl.BlockSpec((TOP_K, P_BLOCK), lambda i: (0, i)),
        out_shape=jax.ShapeDtypeStruct((TOP_K, N_POINTS), jnp.int32),
    )(x, cluster_centers)

    idx_flat = jnp.reshape(jnp.transpose(indices), (OUT_ROWS,))
    out = _sc_gather(x, idx_flat)
    return jnp.reshape(out, (1, OUT_ROWS, D_MODEL))
